# baseline (speedup 1.0000x reference)
import functools

import jax
import jax.numpy as jnp
from jax import lax
from jax.experimental import pallas as pl
from jax.experimental.pallas import tpu as pltpu
from jax.experimental.pallas import tpu_sc as plsc

B = 4096
K = 16
NC = 2
NS = 16
NW = NC * NS
BPW = B // NW
CH = BPW * K // 128


def _rf_body(x_hbm, tab_hbm,
             ent1_out, rel1_out, ent2_out, rel2_out,
             idx_v, idxe_v, idxr_v, ent1_v, rel1_v,
             idx2e_v, idx2r_v, ent2_v, rel2_v,
             sem_e1, sem_r1, sem_e2, sem_r2, sem_w):
    wid = lax.axis_index("s") * NC + lax.axis_index("c")
    base = wid * BPW
    pltpu.sync_copy(x_hbm.at[pl.ds(base, BPW)], idx_v)
    for t in range(BPW // K):
        v2 = idx_v[pl.ds(t * K, K)]
        v2 = v2 + v2
        idxe_v[pl.ds(t * K, K)] = v2
        idxr_v[pl.ds(t * K, K)] = v2 + 1
    ce1 = pltpu.async_copy(tab_hbm.at[idxe_v], ent1_v, sem_e1)
    cr1 = pltpu.async_copy(tab_hbm.at[idxr_v], rel1_v, sem_r1)
    ce1.wait()
    lane = lax.iota(jnp.int32, K)
    perm_lo = lane & 7
    perm_hi = perm_lo + 8
    mask_lo = lane < 8
    for j in range(CH):
        for t in range(4):
            va = ent1_v[j * 8 + 2 * t]
            vb = ent1_v[j * 8 + 2 * t + 1]
            lo = jnp.where(mask_lo, va, jnp.take_along_axis(vb, perm_lo, axis=0))
            hi = jnp.where(mask_lo, jnp.take_along_axis(va, perm_hi, axis=0), vb)
            lo = lo + lo
            hi = hi + hi
            idx2e_v[j, pl.ds(t * K, K)] = lo
            idx2e_v[j, pl.ds(64 + t * K, K)] = hi
            idx2r_v[j, pl.ds(t * K, K)] = lo + 1
            idx2r_v[j, pl.ds(64 + t * K, K)] = hi + 1
    e2 = [pltpu.async_copy(tab_hbm.at[idx2e_v.at[j]], ent2_v.at[j], sem_e2)
          for j in range(CH)]
    r2 = [pltpu.async_copy(tab_hbm.at[idx2r_v.at[j]], rel2_v.at[j], sem_r2)
          for j in range(CH)]
    we1 = pltpu.async_copy(ent1_v, ent1_out.at[pl.ds(base, BPW)], sem_w)
    cr1.wait()
    wr1 = pltpu.async_copy(rel1_v, rel1_out.at[pl.ds(base, BPW)], sem_w)
    ws = []
    for j in range(CH):
        e2[j].wait()
        ws.append(pltpu.async_copy(ent2_v.at[j], ent2_out.at[wid, j], sem_w))
        r2[j].wait()
        ws.append(pltpu.async_copy(rel2_v.at[j], rel2_out.at[wid, j], sem_w))
    we1.wait()
    wr1.wait()
    for c in ws:
        c.wait()


@functools.partial(
    pl.kernel,
    mesh=plsc.VectorSubcoreMesh(core_axis_name="c", subcore_axis_name="s"),
    compiler_params=pltpu.CompilerParams(use_tc_tiling_on_sc=False),
    out_type=[
        jax.ShapeDtypeStruct((B, K), jnp.int32),
        jax.ShapeDtypeStruct((B, K), jnp.int32),
        jax.ShapeDtypeStruct((NW, CH, 128, K), jnp.int32),
        jax.ShapeDtypeStruct((NW, CH, 128, K), jnp.int32),
    ],
    scratch_types=[
        pltpu.VMEM((BPW,), jnp.int32),
        pltpu.VMEM((BPW,), jnp.int32),
        pltpu.VMEM((BPW,), jnp.int32),
        pltpu.VMEM((BPW, K), jnp.int32),
        pltpu.VMEM((BPW, K), jnp.int32),
        pltpu.VMEM((CH, 128), jnp.int32),
        pltpu.VMEM((CH, 128), jnp.int32),
        pltpu.VMEM((CH, 128, K), jnp.int32),
        pltpu.VMEM((CH, 128, K), jnp.int32),
        pltpu.SemaphoreType.DMA,
        pltpu.SemaphoreType.DMA,
        pltpu.SemaphoreType.DMA,
        pltpu.SemaphoreType.DMA,
        pltpu.SemaphoreType.DMA,
    ],
)
def _rf_call(x_hbm, tab_hbm,
             ent1_out, rel1_out, ent2_out, rel2_out,
             idx_v, idxe_v, idxr_v, ent1_v, rel1_v,
             idx2e_v, idx2r_v, ent2_v, rel2_v,
             sem_e1, sem_r1, sem_e2, sem_r2, sem_w):
    _rf_body(x_hbm, tab_hbm,
             ent1_out, rel1_out, ent2_out, rel2_out,
             idx_v, idxe_v, idxr_v, ent1_v, rel1_v,
             idx2e_v, idx2r_v, ent2_v, rel2_v,
             sem_e1, sem_r1, sem_e2, sem_r2, sem_w)


def _untile(o):
    return o.reshape(512, 2, 8, 128).transpose(0, 2, 1, 3).reshape(B, K * K)


def kernel(x, adj_entity, adj_relation):
    out_dtype = adj_entity.dtype
    xi = x.reshape(B).astype(jnp.int32)
    tab = jnp.concatenate(
        [adj_entity.astype(jnp.int32), adj_relation.astype(jnp.int32)],
        axis=1).reshape(2 * 100000, K)
    ent1, rel1, ent2, rel2 = _rf_call(xi, tab)
    return (
        x,
        ent1.astype(out_dtype),
        _untile(ent2).astype(out_dtype),
        rel1.astype(out_dtype),
        _untile(rel2).astype(out_dtype),
    )


# speedup vs baseline: 1.0562x; 1.0562x over previous
import functools

import jax
import jax.numpy as jnp
from jax import lax
from jax.experimental import pallas as pl
from jax.experimental.pallas import tpu as pltpu
from jax.experimental.pallas import tpu_sc as plsc

B = 4096
K = 16
NC = 2
NS = 16
NW = NC * NS
BPW = B // NW
CH = BPW * K // 128


def _transpose16(rows, lane):
    v = list(rows)
    for h in (8, 4, 2, 1):
        pm = (lane - h) & 15
        pp = (lane + h) & 15
        keep = (lane & h) == 0
        nv = list(v)
        for i in range(16):
            if i & h:
                continue
            a, b = v[i], v[i + h]
            nv[i] = jnp.where(keep, a, jnp.take_along_axis(b, pm, axis=0))
            nv[i + h] = jnp.where(keep, jnp.take_along_axis(a, pp, axis=0), b)
        v = nv
    return v


def _rf_body(x_hbm, tab_hbm,
             ent1_out, rel1_out, ent2_out, rel2_out,
             idx_v, idxe_v, idxr_v, ent1_v, rel1_v, e1t_v, r1t_v,
             idx2e_v, idx2r_v, ent2_v, rel2_v,
             sem_e1, sem_r1, sem_e2, sem_r2, sem_w):
    wid = lax.axis_index("s") * NC + lax.axis_index("c")
    base = wid * BPW
    pltpu.sync_copy(x_hbm.at[pl.ds(base, BPW)], idx_v)
    for t in range(BPW // K):
        v2 = idx_v[pl.ds(t * K, K)]
        v2 = v2 + v2
        idxe_v[pl.ds(t * K, K)] = v2
        idxr_v[pl.ds(t * K, K)] = v2 + 1
    ce1 = pltpu.async_copy(tab_hbm.at[idxe_v], ent1_v, sem_e1)
    cr1 = pltpu.async_copy(tab_hbm.at[idxr_v], rel1_v, sem_r1)
    ce1.wait()
    lane = lax.iota(jnp.int32, K)
    perm_lo = lane & 7
    perm_hi = perm_lo + 8
    mask_lo = lane < 8
    erows = [ent1_v[r] for r in range(BPW)]
    for j in range(CH):
        for t in range(4):
            va = erows[j * 8 + 2 * t]
            vb = erows[j * 8 + 2 * t + 1]
            lo = jnp.where(mask_lo, va, jnp.take_along_axis(vb, perm_lo, axis=0))
            hi = jnp.where(mask_lo, jnp.take_along_axis(va, perm_hi, axis=0), vb)
            lo = lo + lo
            hi = hi + hi
            idx2e_v[j, pl.ds(t * K, K)] = lo
            idx2e_v[j, pl.ds(64 + t * K, K)] = hi
            idx2r_v[j, pl.ds(t * K, K)] = lo + 1
            idx2r_v[j, pl.ds(64 + t * K, K)] = hi + 1
    e2 = [pltpu.async_copy(tab_hbm.at[idx2e_v.at[j]], ent2_v.at[j], sem_e2)
          for j in range(CH)]
    r2 = [pltpu.async_copy(tab_hbm.at[idx2r_v.at[j]], rel2_v.at[j], sem_r2)
          for j in range(CH)]
    for b in range(8):
        t = _transpose16([erows[16 * b + i] for i in range(16)], lane)
        for n in range(K):
            e1t_v[n // 8, n % 8, pl.ds(16 * b, K)] = t[n]
    w1 = [pltpu.async_copy(e1t_v.at[tr], ent1_out.at[tr, wid], sem_w)
          for tr in range(2)]
    cr1.wait()
    for b in range(8):
        t = _transpose16([rel1_v[16 * b + i] for i in range(16)], lane)
        for n in range(K):
            r1t_v[n // 8, n % 8, pl.ds(16 * b, K)] = t[n]
    w1 += [pltpu.async_copy(r1t_v.at[tr], rel1_out.at[tr, wid], sem_w)
           for tr in range(2)]
    ws = []
    for j in range(CH):
        e2[j].wait()
        ws.append(pltpu.async_copy(ent2_v.at[j], ent2_out.at[wid, j], sem_w))
        r2[j].wait()
        ws.append(pltpu.async_copy(rel2_v.at[j], rel2_out.at[wid, j], sem_w))
    for c in w1:
        c.wait()
    for c in ws:
        c.wait()


@functools.partial(
    pl.kernel,
    mesh=plsc.VectorSubcoreMesh(core_axis_name="c", subcore_axis_name="s"),
    compiler_params=pltpu.CompilerParams(use_tc_tiling_on_sc=False),
    out_type=[
        jax.ShapeDtypeStruct((2, NW, 8, 128), jnp.int32),
        jax.ShapeDtypeStruct((2, NW, 8, 128), jnp.int32),
        jax.ShapeDtypeStruct((NW, CH, 128, K), jnp.int32),
        jax.ShapeDtypeStruct((NW, CH, 128, K), jnp.int32),
    ],
    scratch_types=[
        pltpu.VMEM((BPW,), jnp.int32),
        pltpu.VMEM((BPW,), jnp.int32),
        pltpu.VMEM((BPW,), jnp.int32),
        pltpu.VMEM((BPW, K), jnp.int32),
        pltpu.VMEM((BPW, K), jnp.int32),
        pltpu.VMEM((2, 8, 128), jnp.int32),
        pltpu.VMEM((2, 8, 128), jnp.int32),
        pltpu.VMEM((CH, 128), jnp.int32),
        pltpu.VMEM((CH, 128), jnp.int32),
        pltpu.VMEM((CH, 128, K), jnp.int32),
        pltpu.VMEM((CH, 128, K), jnp.int32),
        pltpu.SemaphoreType.DMA,
        pltpu.SemaphoreType.DMA,
        pltpu.SemaphoreType.DMA,
        pltpu.SemaphoreType.DMA,
        pltpu.SemaphoreType.DMA,
    ],
)
def _rf_call(x_hbm, tab_hbm,
             ent1_out, rel1_out, ent2_out, rel2_out,
             idx_v, idxe_v, idxr_v, ent1_v, rel1_v, e1t_v, r1t_v,
             idx2e_v, idx2r_v, ent2_v, rel2_v,
             sem_e1, sem_r1, sem_e2, sem_r2, sem_w):
    _rf_body(x_hbm, tab_hbm,
             ent1_out, rel1_out, ent2_out, rel2_out,
             idx_v, idxe_v, idxr_v, ent1_v, rel1_v, e1t_v, r1t_v,
             idx2e_v, idx2r_v, ent2_v, rel2_v,
             sem_e1, sem_r1, sem_e2, sem_r2, sem_w)


def _untile2(o):
    return o.reshape(512, 2, 8, 128).transpose(0, 2, 1, 3).reshape(B, K * K)


def _untile1(o):
    return o.transpose(1, 3, 0, 2).reshape(B, K)


def kernel(x, adj_entity, adj_relation):
    out_dtype = adj_entity.dtype
    xi = x.reshape(B).astype(jnp.int32)
    tab = jnp.concatenate(
        [adj_entity.astype(jnp.int32), adj_relation.astype(jnp.int32)],
        axis=1).reshape(2 * 100000, K)
    ent1, rel1, ent2, rel2 = _rf_call(xi, tab)
    return (
        x,
        _untile1(ent1).astype(out_dtype),
        _untile2(ent2).astype(out_dtype),
        _untile1(rel1).astype(out_dtype),
        _untile2(rel2).astype(out_dtype),
    )
